# baseline (speedup 1.0000x reference)
import jax
import jax.numpy as jnp
from jax import lax
from jax.experimental import pallas as pl
from jax.experimental.pallas import tpu as pltpu
from jax.experimental.pallas import tpu_sc as plsc

BATCH = 4
SEQ = 4096
DIM = 2048
NTOK = BATCH * SEQ
NC = 2
NS = 16
NW = NC * NS
PER_W = NTOK // NW
WPR = SEQ // PER_W
LANES = 16
SLAB = SEQ // NW
SB = 16
NSB = SLAB // SB
NVREG = SEQ // LANES
NPB = PER_W // SB + 1
BIG = jnp.int32(1 << 30)


def _body(x_hbm, embed_hbm, out_hbm, x_v, spos_v, spos2_v, ppos_v, ppos2_v,
          sidx_v, pidx_v, sbufs, pbuf_v, gsems, wsems, psem):
    wid = lax.axis_index("s") * NC + lax.axis_index("c")
    lo = wid * SLAB + 1
    iota = jnp.arange(LANES, dtype=jnp.int32)

    pltpu.sync_copy(x_hbm, x_v)
    for k in range(NSB):
        sidx_v[pl.ds(k * LANES, LANES)] = lo + k * LANES + iota

    def scan_body(j, carry):
        cs_, wps, jfps = carry
        cs_, wps, jfps = list(cs_), list(wps), list(jfps)
        for b in range(BATCH):
            v = x_v[pl.ds(b * SEQ + j * LANES, LANES)]
            ones = jnp.where(v != 0, 1, 0).astype(jnp.int32)
            cum = jnp.cumsum(ones) + cs_[b]
            pos = cum * ones
            m = (pos >= lo) & (pos < lo + SLAB)
            dst = b * SEQ + j * LANES + iota
            plsc.store_compressed(
                spos_v.at[pl.ds(wps[b], LANES)], dst, mask=m)
            s = jnp.sum(ones)
            jfps[b] = jnp.minimum(jfps[b], jnp.where(s < LANES, j, BIG))
            cs_[b] = cs_[b] + s
            wps[b] = wps[b] + jnp.sum(m.astype(jnp.int32))
        return tuple(cs_), tuple(wps), tuple(jfps)

    _, wp_ends, jfp_ends = lax.fori_loop(
        0, NVREG, scan_body,
        (
            tuple(jnp.int32(0) for _ in range(BATCH)),
            tuple(jnp.int32(b * SLAB) for b in range(BATCH)),
            tuple(BIG for _ in range(BATCH)),
        ),
    )

    nrows = []
    for b in range(BATCH):
        wp_end = wp_ends[b]
        nrows.append(wp_end - b * SLAB)
        jj = jnp.minimum(jfp_ends[b], NVREG - 1)
        pv = x_v[pl.ds(b * SEQ + jj * LANES, LANES)]
        dstv = b * SEQ + jj * LANES + iota
        fp_b = jnp.min(jnp.where(pv == 0, dstv, BIG))
        m_b = nrows[b]
        for k in range(SLAB // LANES):
            off = b * SLAB + k * LANES
            old = spos_v[pl.ds(off, LANES)]
            keep = (k * LANES + iota) < m_b
            spos_v[pl.ds(off, LANES)] = jnp.where(keep, old, fp_b)

    for r in range(BATCH * NSB):
        spos2_v[r, :] = spos_v[pl.ds(r * LANES, LANES)]

    wstart = (wid // WPR) * SEQ + (wid % WPR) * PER_W

    def pad_body(j, wp):
        v = x_v[pl.ds(wstart + j * LANES, LANES)]
        pm = v == 0
        dst = wstart + j * LANES + iota
        plsc.store_compressed(ppos_v.at[pl.ds(wp, LANES)], dst, mask=pm)
        return wp + jnp.sum(pm.astype(jnp.int32))

    npad = lax.fori_loop(0, PER_W // LANES, pad_body, jnp.int32(0))

    def start_g(j, slot):
        pltpu.async_copy(embed_hbm.at[sidx_v.at[pl.ds(j * SB, SB)]],
                         sbufs[slot], gsems[slot])

    def wait_g(slot):
        pltpu.make_async_copy(embed_hbm.at[sidx_v.at[pl.ds(0, SB)]],
                              sbufs[slot], gsems[slot]).wait()

    def drain_scatters(sem, n):
        def d(i, _):
            pltpu.make_async_copy(sbufs[0], out_hbm.at[spos2_v.at[0]],
                                  sem).wait()
            return 0

        lax.fori_loop(0, n, d, 0)

    def writes_for(j, slot):
        count = jnp.int32(0)
        for b in range(BATCH):
            valid = jnp.clip(nrows[b] - SB * j, 0, SB)
            full = valid == SB
            partial = jnp.logical_and(valid > 0, valid < SB)
            r = b * NSB + j

            @pl.when(full)
            def _():
                pltpu.async_copy(sbufs[slot], out_hbm.at[spos2_v.at[r]],
                                 wsems[slot])

            @pl.when(partial)
            def _():
                pidx_v[...] = jnp.where(iota < valid, lo + j * SB + iota, 0)
                pltpu.sync_copy(embed_hbm.at[pidx_v], pbuf_v)
                pltpu.sync_copy(pbuf_v, out_hbm.at[spos2_v.at[r]])

            count = count + full.astype(jnp.int32)
        return count

    start_g(0, 0)
    hist = []
    for j in range(NSB):
        slot = j % 2
        if j + 1 < NSB:
            if j >= 1:
                drain_scatters(wsems[(j + 1) % 2], hist[j - 1])
            start_g(j + 1, (j + 1) % 2)
        wait_g(slot)
        hist.append(writes_for(j, slot))
    drain_scatters(wsems[(NSB - 2) % 2], hist[NSB - 2])
    drain_scatters(wsems[(NSB - 1) % 2], hist[NSB - 1])

    @pl.when(npad > 0)
    def _():
        pv0 = ppos_v[pl.ds(0, LANES)]
        fp_own = jnp.sum(pv0 * (iota == 0))
        ppos_v[pl.ds(npad, LANES)] = jnp.broadcast_to(
            fp_own, (LANES,)).astype(jnp.int32)

        def stage(i, _):
            ppos2_v[i, :] = ppos_v[pl.ds(i * LANES, LANES)]
            return 0

        nblk = (npad + SB - 1) // SB
        lax.fori_loop(0, nblk, stage, 0)
        pidx_v[...] = iota * 0
        pltpu.sync_copy(embed_hbm.at[pidx_v], pbuf_v)

        def pw(k, _):
            pltpu.async_copy(pbuf_v, out_hbm.at[ppos2_v.at[k]], psem)
            return 0

        lax.fori_loop(0, nblk, pw, 0)

        def pd(k, _):
            pltpu.make_async_copy(pbuf_v, out_hbm.at[ppos2_v.at[0]],
                                  psem).wait()
            return 0

        lax.fori_loop(0, nblk, pd, 0)


@jax.jit
def kernel(x, embed):
    mesh = plsc.VectorSubcoreMesh(
        core_axis_name="c", subcore_axis_name="s", num_cores=NC,
        num_subcores=NS,
    )
    out = pl.kernel(
        _body,
        out_type=jax.ShapeDtypeStruct((NTOK, DIM), jnp.float32),
        mesh=mesh,
        compiler_params=pltpu.CompilerParams(needs_layout_passes=False),
        scratch_types=[
            pltpu.VMEM((NTOK,), jnp.int32),
            pltpu.VMEM((BATCH * SLAB + LANES,), jnp.int32),
            pltpu.VMEM((BATCH * NSB, LANES), jnp.int32),
            pltpu.VMEM((PER_W + LANES,), jnp.int32),
            pltpu.VMEM((NPB, LANES), jnp.int32),
            pltpu.VMEM((SLAB,), jnp.int32),
            pltpu.VMEM((LANES,), jnp.int32),
            tuple(pltpu.VMEM((SB, DIM), jnp.float32) for _ in range(2)),
            pltpu.VMEM((SB, DIM), jnp.float32),
            tuple(pltpu.SemaphoreType.DMA for _ in range(2)),
            tuple(pltpu.SemaphoreType.DMA for _ in range(2)),
            pltpu.SemaphoreType.DMA,
        ],
    )(x.reshape(NTOK), embed)
    return out.reshape(BATCH, SEQ, DIM)


# speedup vs baseline: 1.0306x; 1.0306x over previous
import jax
import jax.numpy as jnp
from jax import lax
from jax.experimental import pallas as pl
from jax.experimental.pallas import tpu as pltpu
from jax.experimental.pallas import tpu_sc as plsc

BATCH = 4
SEQ = 4096
DIM = 2048
NTOK = BATCH * SEQ
NC = 2
NS = 16
NW = NC * NS
PER_W = NTOK // NW
WPR = SEQ // PER_W
LANES = 16
CH = 16
NCH = PER_W // CH
NB = 3
NVREG = PER_W // LANES


def _body(x_hbm, embed_hbm, out_hbm, x_v, idx_v, rows_bufs, gsems, wsems):
    wid = lax.axis_index("s") * NC + lax.axis_index("c")
    row = wid // WPR
    ch = wid % WPR

    pltpu.sync_copy(x_hbm.at[pl.ds(row * SEQ, SEQ)], x_v)

    def pre_body(i, carry):
        v = x_v[pl.ds(i * LANES, LANES)]
        ones = jnp.where(v != 0, 1, 0).astype(jnp.int32)
        return carry + jnp.sum(ones)

    carry0 = lax.fori_loop(0, ch * NVREG, pre_body, jnp.int32(0))

    base = ch * PER_W

    def pos_body(j, carry):
        v = x_v[pl.ds(base + j * LANES, LANES)]
        ones = jnp.where(v != 0, 1, 0).astype(jnp.int32)
        cs = jnp.cumsum(ones) + carry
        idx_v[pl.ds(j * LANES, LANES)] = cs * ones
        return carry + jnp.sum(ones)

    lax.fori_loop(0, NVREG, pos_body, carry0)

    out_base = wid * PER_W

    def start_gather(g, b):
        pltpu.async_copy(embed_hbm.at[idx_v.at[pl.ds(g * CH, CH)]],
                         rows_bufs[b], gsems[b])

    def wait_gather(b):
        pltpu.make_async_copy(embed_hbm.at[idx_v.at[pl.ds(0, CH)]],
                              rows_bufs[b], gsems[b]).wait()

    def start_write(g, b):
        pltpu.async_copy(rows_bufs[b],
                         out_hbm.at[pl.ds(out_base + g * CH, CH)], wsems[b])

    def wait_write(b):
        pltpu.make_async_copy(rows_bufs[b],
                              out_hbm.at[pl.ds(out_base, CH)],
                              wsems[b]).wait()

    def step(g, b, nxt_b, has_next, wait_prev_write):
        wait_gather(b)
        start_write(g, b)
        if has_next:
            if wait_prev_write:
                wait_write(nxt_b)
            start_gather(g + NB - 1, nxt_b)

    for g in range(NB - 1):
        start_gather(g, g % NB)

    for g in range(NB):
        step(g, g % NB, (g + NB - 1) % NB, True, g + NB - 1 >= NB)

    def g_body(k, _):
        g0 = NB * k
        for j in range(NB):
            step(g0 + j, j % NB, (j + NB - 1) % NB, True, True)
        return 0

    n_full = NCH // NB
    lax.fori_loop(1, n_full, g_body, 0)
    for g in range(NB * n_full, NCH):
        has_next = g + NB - 1 < NCH
        step(g, g % NB, (g + NB - 1) % NB, has_next, has_next)
    for b in range(NB):
        wait_write(b)


@jax.jit
def kernel(x, embed):
    x_flat = x.reshape(NTOK)
    mesh = plsc.VectorSubcoreMesh(
        core_axis_name="c", subcore_axis_name="s", num_cores=NC,
        num_subcores=NS,
    )
    out = pl.kernel(
        _body,
        out_type=jax.ShapeDtypeStruct((NTOK, DIM), jnp.float32),
        mesh=mesh,
        compiler_params=pltpu.CompilerParams(needs_layout_passes=False),
        scratch_types=[
            pltpu.VMEM((SEQ,), jnp.int32),
            pltpu.VMEM((PER_W,), jnp.int32),
            tuple(pltpu.VMEM((CH, DIM), jnp.float32) for _ in range(NB)),
            tuple(pltpu.SemaphoreType.DMA for _ in range(NB)),
            tuple(pltpu.SemaphoreType.DMA for _ in range(NB)),
        ],
    )(x_flat, embed)
    return out.reshape(BATCH, SEQ, DIM)
